# Initial kernel scaffold; baseline (speedup 1.0000x reference)
import jax
import jax.numpy as jnp
from jax import lax
from jax.experimental import pallas as pl
from jax.experimental.pallas import tpu as pltpu
from jax.experimental.pallas import tpu_sc as plsc

NW = 32
CH = 128
L = 16


def _body(table, offs, out, off_v, out_v, sem, *bufs):
    idx_b = bufs[0:8]
    w_b = bufs[8:16]
    r_b = bufs[16:24]

    pw = out.shape[0] // (8 * NW)
    nchunk = pw // CH
    n_per_batch = 96 * 96 * 96

    wid = lax.axis_index("s") * 2 + lax.axis_index("c")
    pbase = wid * pw
    vbase = (pbase // (out.shape[0] // (8 * 2))) * n_per_batch

    pltpu.sync_copy(offs.at[pl.ds(pbase * 3, pw * 3)], off_v)

    lanes = lax.iota(jnp.int32, L)
    sel = (lanes >= 8).astype(jnp.int32)
    colid = lanes - 8 * sel

    def chunk(ci, carry):
        cbase = ci * CH
        for i in range(CH // L):
            p = cbase + i * L + lanes
            fo = p * 3
            yc = plsc.load_gather(off_v, [fo])
            xc = plsc.load_gather(off_v, [fo + 1])
            zc = plsc.load_gather(off_v, [fo + 2])
            xi = xc.astype(jnp.int32)
            yi = yc.astype(jnp.int32)
            zi = zc.astype(jnp.int32)
            fx = xc - xi.astype(jnp.float32)
            fy = yc - yi.astype(jnp.float32)
            fz = zc - zi.astype(jnp.float32)
            gx = 1.0 - fx
            gy = 1.0 - fy
            gz = 1.0 - fz
            v = vbase + yi * 9216 + xi * 96 + zi
            zx00 = gz * gx
            zx01 = gz * fx
            zx10 = fz * gx
            zx11 = fz * fx
            sl = pl.ds(i * L, L)
            idx_b[0][sl] = v
            w_b[0][sl] = zx00 * gy
            idx_b[1][sl] = v + 9216
            w_b[1][sl] = zx00 * fy
            idx_b[2][sl] = v + 96
            w_b[2][sl] = zx01 * gy
            idx_b[3][sl] = v + 9312
            w_b[3][sl] = zx01 * fy
            idx_b[4][sl] = v + 1
            w_b[4][sl] = zx10 * gy
            idx_b[5][sl] = v + 9217
            w_b[5][sl] = zx10 * fy
            idx_b[6][sl] = v + 97
            w_b[6][sl] = zx11 * gy
            idx_b[7][sl] = v + 9313
            w_b[7][sl] = zx11 * fy

        copies = [pltpu.async_copy(table.at[idx_b[c]], r_b[c], sem)
                  for c in range(8)]
        for cp in copies:
            cp.wait()

        for j in range(CH // 2):
            rvec = 2 * j + sel
            acc = plsc.load_gather(w_b[0], [rvec]) * plsc.load_gather(
                r_b[0], [rvec, colid])
            for c in range(1, 8):
                acc = acc + plsc.load_gather(w_b[c], [rvec]) * plsc.load_gather(
                    r_b[c], [rvec, colid])
            out_v[pl.ds(j * L, L)] = acc

        pltpu.sync_copy(out_v, out.at[pl.ds((pbase + cbase) * 8, CH * 8)])
        return carry

    lax.fori_loop(0, nchunk, chunk, 0)


def kernel(im, offsets):
    B, H, W, D, C = im.shape
    N = offsets.shape[1]
    table = im.reshape(B * H * W * D, C)
    offs = offsets.reshape(B * N * 3)

    mesh = plsc.VectorSubcoreMesh(core_axis_name="c", subcore_axis_name="s")
    pw = B * N // NW
    scratch = [
        pltpu.VMEM((pw * 3,), jnp.float32),
        pltpu.VMEM((CH * 8,), jnp.float32),
        pltpu.SemaphoreType.DMA,
    ]
    scratch += [pltpu.VMEM((CH,), jnp.int32) for _ in range(8)]
    scratch += [pltpu.VMEM((CH,), jnp.float32) for _ in range(8)]
    scratch += [pltpu.VMEM((CH, C), jnp.float32) for _ in range(8)]

    run = pl.kernel(
        _body,
        out_type=jax.ShapeDtypeStruct((B * N * C,), jnp.float32),
        mesh=mesh,
        scratch_types=scratch,
    )
    return run(table, offs).reshape(B, N, C)


# speedup vs baseline: 1.9664x; 1.9664x over previous
import jax
import jax.numpy as jnp
from jax import lax
from jax.experimental import pallas as pl
from jax.experimental.pallas import tpu as pltpu
from jax.experimental.pallas import tpu_sc as plsc

NW = 32
CH = 128
L = 16


def _body(table, offs, out, off_v, out_v, sem, *bufs):
    idx_b = bufs[0:8]
    w_b = bufs[8:16]
    r_b = bufs[16:24]

    pw = out.shape[0] // (8 * NW)
    nchunk = pw // CH
    n_per_batch = 96 * 96 * 96

    wid = lax.axis_index("s") * 2 + lax.axis_index("c")
    pbase = wid * pw
    vbase = (pbase // (out.shape[0] // (8 * 2))) * n_per_batch

    pltpu.sync_copy(offs.at[pl.ds(pbase * 3, pw * 3)], off_v)

    def chunk(ci, carry):
        lanes = lax.iota(jnp.int32, L)
        sel = lanes >> 3
        colid = lanes & 7
        cbase = ci * CH
        for i in range(CH // L):
            p = cbase + i * L + lanes
            fo = p * 3
            yc = plsc.load_gather(off_v, [fo])
            xc = plsc.load_gather(off_v, [fo + 1])
            zc = plsc.load_gather(off_v, [fo + 2])
            xi = xc.astype(jnp.int32)
            yi = yc.astype(jnp.int32)
            zi = zc.astype(jnp.int32)
            fx = xc - xi.astype(jnp.float32)
            fy = yc - yi.astype(jnp.float32)
            fz = zc - zi.astype(jnp.float32)
            gx = 1.0 - fx
            gy = 1.0 - fy
            gz = 1.0 - fz
            v = vbase + yi * 9216 + xi * 96 + zi
            zx00 = gz * gx
            zx01 = gz * fx
            zx10 = fz * gx
            zx11 = fz * fx
            sl = pl.ds(i * L, L)
            idx_b[0][sl] = v
            w_b[0][sl] = zx00 * gy
            idx_b[1][sl] = v + 9216
            w_b[1][sl] = zx00 * fy
            idx_b[2][sl] = v + 96
            w_b[2][sl] = zx01 * gy
            idx_b[3][sl] = v + 9312
            w_b[3][sl] = zx01 * fy
            idx_b[4][sl] = v + 1
            w_b[4][sl] = zx10 * gy
            idx_b[5][sl] = v + 9217
            w_b[5][sl] = zx10 * fy
            idx_b[6][sl] = v + 97
            w_b[6][sl] = zx11 * gy
            idx_b[7][sl] = v + 9313
            w_b[7][sl] = zx11 * fy

        copies = [pltpu.async_copy(table.at[idx_b[c]], r_b[c], sem)
                  for c in range(8)]
        for cp in copies:
            cp.wait()

        for j in range(CH // 2):
            rvec = 2 * j + sel
            acc = plsc.load_gather(w_b[0], [rvec]) * plsc.load_gather(
                r_b[0], [rvec, colid])
            for c in range(1, 8):
                acc = acc + plsc.load_gather(w_b[c], [rvec]) * plsc.load_gather(
                    r_b[c], [rvec, colid])
            out_v[pl.ds(j * L, L)] = acc

        pltpu.sync_copy(out_v, out.at[pl.ds((pbase + cbase) * 8, CH * 8)])
        return carry

    lax.fori_loop(0, nchunk, chunk, 0)


def kernel(im, offsets):
    B, H, W, D, C = im.shape
    N = offsets.shape[1]
    table = im.reshape(B * H * W * D, C)
    offs = offsets.reshape(B * N * 3)

    mesh = plsc.VectorSubcoreMesh(core_axis_name="c", subcore_axis_name="s")
    pw = B * N // NW
    scratch = [
        pltpu.VMEM((pw * 3,), jnp.float32),
        pltpu.VMEM((CH * 8,), jnp.float32),
        pltpu.SemaphoreType.DMA,
    ]
    scratch += [pltpu.VMEM((CH,), jnp.int32) for _ in range(8)]
    scratch += [pltpu.VMEM((CH,), jnp.float32) for _ in range(8)]
    scratch += [pltpu.VMEM((CH, C), jnp.float32) for _ in range(8)]

    run = pl.kernel(
        _body,
        out_type=jax.ShapeDtypeStruct((B * N * C,), jnp.float32),
        mesh=mesh,
        scratch_types=scratch,
        compiler_params=pltpu.CompilerParams(
            needs_layout_passes=False, use_tc_tiling_on_sc=False),
    )
    return run(table, offs).reshape(B, N, C)


# speedup vs baseline: 2.0856x; 1.0606x over previous
import jax
import jax.numpy as jnp
from jax import lax
from jax.experimental import pallas as pl
from jax.experimental.pallas import tpu as pltpu
from jax.experimental.pallas import tpu_sc as plsc

NW = 32
CH = 128
L = 16
NBUF = 19


def _body(table, offs, out, off_v, out_v, sem0, sem1, *bufs):
    sets = []
    for s in range(2):
        grp = bufs[s * NBUF:(s + 1) * NBUF]
        sets.append((grp[0:8], grp[8:11], grp[11:19]))

    pw = out.shape[0] // (8 * NW)
    nchunk = pw // CH
    n_per_batch = 96 * 96 * 96

    wid = lax.axis_index("s") * 2 + lax.axis_index("c")
    pbase = wid * pw
    vbase = (pbase // (out.shape[0] // (8 * 2))) * n_per_batch

    pltpu.sync_copy(offs.at[pl.ds(pbase * 3, pw * 3)], off_v)

    voff = (0, 9216, 96, 9312, 1, 9217, 97, 9313)

    def phase_a(cbase, st):
        idx_b, f_b, _ = st
        lanes = lax.iota(jnp.int32, L)
        for i in range(CH // L):
            fo = (cbase + i * L + lanes) * 3
            yc = plsc.load_gather(off_v, [fo])
            xc = plsc.load_gather(off_v, [fo + 1])
            zc = plsc.load_gather(off_v, [fo + 2])
            xi = xc.astype(jnp.int32)
            yi = yc.astype(jnp.int32)
            zi = zc.astype(jnp.int32)
            sl = pl.ds(i * L, L)
            f_b[0][sl] = zc - zi.astype(jnp.float32)
            f_b[1][sl] = xc - xi.astype(jnp.float32)
            f_b[2][sl] = yc - yi.astype(jnp.float32)
            v = vbase + yi * 9216 + xi * 96 + zi
            for c in range(8):
                idx_b[c][sl] = v + voff[c]

    def fire(st, sem):
        idx_b, _, r_b = st
        for c in range(8):
            pltpu.async_copy(table.at[idx_b[c]], r_b[c], sem)

    def drain(st, sem):
        idx_b, _, r_b = st
        for c in range(8):
            pltpu.make_async_copy(table.at[idx_b[c]], r_b[c], sem).wait()

    def blend(cbase, st):
        _, f_b, r_b = st
        lanes = lax.iota(jnp.int32, L)
        sel = lanes >> 3
        colid = lanes & 7
        for j in range(CH // 2):
            rvec = 2 * j + sel
            fz = plsc.load_gather(f_b[0], [rvec])
            fx = plsc.load_gather(f_b[1], [rvec])
            fy = plsc.load_gather(f_b[2], [rvec])
            i0 = plsc.load_gather(r_b[0], [rvec, colid])
            i1 = plsc.load_gather(r_b[1], [rvec, colid])
            i2 = plsc.load_gather(r_b[2], [rvec, colid])
            i3 = plsc.load_gather(r_b[3], [rvec, colid])
            i4 = plsc.load_gather(r_b[4], [rvec, colid])
            i5 = plsc.load_gather(r_b[5], [rvec, colid])
            i6 = plsc.load_gather(r_b[6], [rvec, colid])
            i7 = plsc.load_gather(r_b[7], [rvec, colid])
            a = i0 + fz * (i4 - i0)
            b = i1 + fz * (i5 - i1)
            c = i2 + fz * (i6 - i2)
            d = i3 + fz * (i7 - i3)
            e = a + fx * (c - a)
            f = b + fx * (d - b)
            out_v[pl.ds(j * L, L)] = e + fy * (f - e)
        pltpu.sync_copy(out_v, out.at[pl.ds((pbase + cbase) * 8, CH * 8)])

    phase_a(0, sets[0])
    fire(sets[0], sem0)

    def pair(i, carry):
        cb0 = (2 * i) * CH
        cb1 = cb0 + CH
        cb2 = cb0 + 2 * CH
        phase_a(cb1, sets[1])
        fire(sets[1], sem1)
        drain(sets[0], sem0)
        blend(cb0, sets[0])

        @pl.when(cb2 < pw)
        def _():
            phase_a(cb2, sets[0])
            fire(sets[0], sem0)

        drain(sets[1], sem1)
        blend(cb1, sets[1])
        return carry

    lax.fori_loop(0, nchunk // 2, pair, 0)


def kernel(im, offsets):
    B, H, W, D, C = im.shape
    N = offsets.shape[1]
    table = im.reshape(B * H * W * D, C)
    offs = offsets.reshape(B * N * 3)

    mesh = plsc.VectorSubcoreMesh(core_axis_name="c", subcore_axis_name="s")
    pw = B * N // NW
    scratch = [
        pltpu.VMEM((pw * 3,), jnp.float32),
        pltpu.VMEM((CH * 8,), jnp.float32),
        pltpu.SemaphoreType.DMA,
        pltpu.SemaphoreType.DMA,
    ]
    for _ in range(2):
        scratch += [pltpu.VMEM((CH,), jnp.int32) for _ in range(8)]
        scratch += [pltpu.VMEM((CH,), jnp.float32) for _ in range(3)]
        scratch += [pltpu.VMEM((CH, C), jnp.float32) for _ in range(8)]

    run = pl.kernel(
        _body,
        out_type=jax.ShapeDtypeStruct((B * N * C,), jnp.float32),
        mesh=mesh,
        scratch_types=scratch,
        compiler_params=pltpu.CompilerParams(
            needs_layout_passes=False, use_tc_tiling_on_sc=False),
    )
    return run(table, offs).reshape(B, N, C)
